# Initial kernel scaffold; baseline (speedup 1.0000x reference)
import jax
import jax.numpy as jnp
from jax.experimental import pallas as pl


def kernel(tensor, bank_selections, bank_probabilities, weights, bias):
    raise NotImplementedError("write your pallas kernel here")


# speedup vs baseline: 3.1772x; 3.1772x over previous
import jax
import jax.numpy as jnp
from jax.experimental import pallas as pl
from jax.experimental.pallas import tpu as pltpu

N = 2048
IN_FEATURES = 128
OUT_FEATURES = 128
NUM_BANKS = 64
TOP_K = 2


def _banked_kernel(sel_ref, prob_ref, x_ref, w_ref, b_ref, out_ref):
    j = pl.program_id(0)

    @pl.when(j == 0)
    def _():
        out_ref[...] = jnp.zeros_like(out_ref)

    sel = sel_ref[...]
    prob = prob_ref[...]
    p = jnp.sum(jnp.where(sel == j, prob, 0.0), axis=1)
    x = x_ref[...] * p[:, None]
    w = w_ref[0]
    acc = jnp.dot(x, w, preferred_element_type=jnp.float32)
    acc = acc + p[:, None] * b_ref[0]
    out_ref[...] += acc


def kernel(tensor, bank_selections, bank_probabilities, weights, bias):
    sel = bank_selections.astype(jnp.int32)
    out = pl.pallas_call(
        _banked_kernel,
        grid=(NUM_BANKS,),
        in_specs=[
            pl.BlockSpec((N, TOP_K), lambda j: (0, 0)),
            pl.BlockSpec((N, TOP_K), lambda j: (0, 0)),
            pl.BlockSpec((N, IN_FEATURES), lambda j: (0, 0)),
            pl.BlockSpec((1, IN_FEATURES, OUT_FEATURES), lambda j: (j, 0, 0)),
            pl.BlockSpec((1, 1, OUT_FEATURES), lambda j: (j, 0, 0)),
        ],
        out_specs=pl.BlockSpec((N, OUT_FEATURES), lambda j: (0, 0)),
        out_shape=jax.ShapeDtypeStruct((N, OUT_FEATURES), jnp.float32),
        compiler_params=pltpu.CompilerParams(
            dimension_semantics=("arbitrary",),
        ),
    )(sel, bank_probabilities, tensor, weights,
      bias.reshape(NUM_BANKS, 1, OUT_FEATURES))
    return out


# speedup vs baseline: 3.7704x; 1.1867x over previous
import jax
import jax.numpy as jnp
from jax.experimental import pallas as pl
from jax.experimental.pallas import tpu as pltpu

N = 2048
IN_FEATURES = 128
OUT_FEATURES = 128
NUM_BANKS = 64
TOP_K = 2
CHUNK = 8
NCHUNKS = NUM_BANKS // CHUNK


def _p_kernel(sel_ref, prob_ref, p_ref):
    c = pl.program_id(0)
    sel = sel_ref[...]
    prob = prob_ref[...]
    banks = jax.lax.broadcasted_iota(jnp.int32, (N, CHUNK), 1) + c * CHUNK
    p = jnp.zeros((N, CHUNK), jnp.float32)
    for k in range(TOP_K):
        p += jnp.where(sel[:, k:k + 1] == banks, prob[:, k:k + 1], 0.0)
    p_ref[0] = p


def _mm_kernel(p_ref, x_ref, w_ref, b_ref, out_ref):
    c = pl.program_id(0)
    p8 = p_ref[0]
    x = x_ref[...]
    acc = jnp.dot(p8, b_ref[...], preferred_element_type=jnp.float32)
    for i in range(CHUNK):
        z = jnp.dot(x, w_ref[i], preferred_element_type=jnp.float32)
        acc = acc + p8[:, i:i + 1] * z

    @pl.when(c == 0)
    def _():
        out_ref[...] = acc

    @pl.when(c != 0)
    def _():
        out_ref[...] += acc


def kernel(tensor, bank_selections, bank_probabilities, weights, bias):
    sel = bank_selections.astype(jnp.int32)
    xb = tensor.astype(jnp.bfloat16)
    wb = weights.astype(jnp.bfloat16)

    p = pl.pallas_call(
        _p_kernel,
        grid=(NCHUNKS,),
        in_specs=[
            pl.BlockSpec((N, TOP_K), lambda c: (0, 0)),
            pl.BlockSpec((N, TOP_K), lambda c: (0, 0)),
        ],
        out_specs=pl.BlockSpec((1, N, CHUNK), lambda c: (c, 0, 0)),
        out_shape=jax.ShapeDtypeStruct((NCHUNKS, N, CHUNK), jnp.float32),
        compiler_params=pltpu.CompilerParams(
            dimension_semantics=("arbitrary",),
        ),
    )(sel, bank_probabilities)

    out = pl.pallas_call(
        _mm_kernel,
        grid=(NCHUNKS,),
        in_specs=[
            pl.BlockSpec((1, N, CHUNK), lambda c: (c, 0, 0)),
            pl.BlockSpec((N, IN_FEATURES), lambda c: (0, 0)),
            pl.BlockSpec((CHUNK, IN_FEATURES, OUT_FEATURES), lambda c: (c, 0, 0)),
            pl.BlockSpec((CHUNK, OUT_FEATURES), lambda c: (c, 0)),
        ],
        out_specs=pl.BlockSpec((N, OUT_FEATURES), lambda c: (0, 0)),
        out_shape=jax.ShapeDtypeStruct((N, OUT_FEATURES), jnp.float32),
        compiler_params=pltpu.CompilerParams(
            dimension_semantics=("arbitrary",),
        ),
    )(p, xb, wb, bias)
    return out


# speedup vs baseline: 4.5343x; 1.2026x over previous
import jax
import jax.numpy as jnp
from jax.experimental import pallas as pl
from jax.experimental.pallas import tpu as pltpu

N = 2048
IN_FEATURES = 128
OUT_FEATURES = 128
NUM_BANKS = 64
TOP_K = 2
TILE = 256
NTILES = N // TILE


def _mm_kernel(sel_ref, prob_ref, x_ref, w_ref, b_ref, out_ref):
    sel = sel_ref[...]
    prob = prob_ref[...]
    banks = jax.lax.broadcasted_iota(jnp.int32, (TILE, NUM_BANKS), 1)
    p = jnp.zeros((TILE, NUM_BANKS), jnp.float32)
    for k in range(TOP_K):
        p += jnp.where(sel[:, k:k + 1] == banks, prob[:, k:k + 1], 0.0)

    x = x_ref[...]
    acc = jnp.dot(p, b_ref[...], preferred_element_type=jnp.float32)
    for b in range(NUM_BANKS):
        z = jnp.dot(x, w_ref[b], preferred_element_type=jnp.float32)
        acc = acc + p[:, b:b + 1] * z
    out_ref[...] = acc


def kernel(tensor, bank_selections, bank_probabilities, weights, bias):
    sel = bank_selections.astype(jnp.int32)
    xb = tensor.astype(jnp.bfloat16)
    wb = weights.astype(jnp.bfloat16)

    out = pl.pallas_call(
        _mm_kernel,
        grid=(NTILES,),
        in_specs=[
            pl.BlockSpec((TILE, TOP_K), lambda t: (t, 0)),
            pl.BlockSpec((TILE, TOP_K), lambda t: (t, 0)),
            pl.BlockSpec((TILE, IN_FEATURES), lambda t: (t, 0)),
            pl.BlockSpec((NUM_BANKS, IN_FEATURES, OUT_FEATURES),
                         lambda t: (0, 0, 0)),
            pl.BlockSpec((NUM_BANKS, OUT_FEATURES), lambda t: (0, 0)),
        ],
        out_specs=pl.BlockSpec((TILE, OUT_FEATURES), lambda t: (t, 0)),
        out_shape=jax.ShapeDtypeStruct((N, OUT_FEATURES), jnp.float32),
        compiler_params=pltpu.CompilerParams(
            dimension_semantics=("parallel",),
        ),
    )(sel, bank_probabilities, xb, wb, bias)
    return out
